# baseline (speedup 1.0000x reference)
import functools

import numpy as np
import jax
import jax.numpy as jnp
from jax import lax
from jax.experimental import pallas as pl
from jax.experimental.pallas import tpu as pltpu
from jax.experimental.pallas import tpu_sc as plsc

_NUM_BINS = 512
_NUM_TILINGS = 16
_BATCH = 16384

_NC = 2
_NS = 16
_L = 16
_NW = _NC * _NS
_SPW = _BATCH // _NW
_GPW = _SPW // _L
_IDX_PER_W = _SPW * _NUM_TILINGS
_CHUNK = 128
_NCHUNK = _IDX_PER_W // _CHUNK

_W = np.float32(6.0) / np.float32(_NUM_BINS)
_RW = np.float32(1.0) / _W
_OFFS = [np.float32(np.float32(t) / np.float32(_NUM_TILINGS)) * _W
         for t in range(_NUM_TILINGS)]

_TSTRIDE = _NUM_BINS * _NUM_BINS
_ASTRIDE = _NUM_TILINGS * _TSTRIDE

_mesh = plsc.VectorSubcoreMesh(core_axis_name="c", subcore_axis_name="s")


@functools.partial(
    pl.kernel,
    mesh=_mesh,
    out_type=jax.ShapeDtypeStruct((2 * _BATCH,), jnp.float32),
    scratch_types=[
        pltpu.VMEM((4 * _SPW,), jnp.float32),
        pltpu.VMEM((_IDX_PER_W,), jnp.int32),
        pltpu.VMEM((_IDX_PER_W,), jnp.float32),
        pltpu.VMEM((_IDX_PER_W,), jnp.float32),
        pltpu.VMEM((2 * _SPW,), jnp.float32),
        pltpu.SemaphoreType.DMA,
    ],
)
def _tile_coding_sc(cols_hbm, wp_hbm, wv_hbm, out_hbm,
                    cols_v, idx_v, gp_v, gv_v, out_v, sem):
    wid = lax.axis_index("s") * _NC + lax.axis_index("c")
    base = wid * _SPW

    for c in range(4):
        pltpu.sync_copy(cols_hbm.at[pl.ds(c * _BATCH + base, _SPW)],
                        cols_v.at[pl.ds(c * _SPW, _SPW)])

    def idx_body(g, carry):
        s = g * _L
        x0 = cols_v[pl.ds(s, _L)]
        x1 = cols_v[pl.ds(_SPW + s, _L)]
        s3 = cols_v[pl.ds(2 * _SPW + s, _L)]
        s4 = cols_v[pl.ds(3 * _SPW + s, _L)]
        abase = (s3 + 2.0 * s4).astype(jnp.int32) * _ASTRIDE
        d0 = x0 + 3.0
        d1 = x1 + 3.0
        for t in range(_NUM_TILINGS):
            q0 = (d0 + _OFFS[t]) * _RW
            q1 = (d1 + _OFFS[t]) * _RW
            i0 = jnp.clip(q0.astype(jnp.int32), 0, _NUM_BINS - 1)
            i1 = jnp.clip(q1.astype(jnp.int32), 0, _NUM_BINS - 1)
            flat = abase + (t * _TSTRIDE) + i0 * _NUM_BINS + i1
            idx_v[pl.ds(t * _SPW + s, _L)] = flat
        return carry

    lax.fori_loop(0, _GPW, idx_body, 0)

    def fire(j, carry):
        c = j * _CHUNK
        idxs = idx_v.at[pl.ds(c, _CHUNK)]
        pltpu.make_async_copy(wp_hbm.at[idxs], gp_v.at[pl.ds(c, _CHUNK)], sem).start()
        pltpu.make_async_copy(wv_hbm.at[idxs], gv_v.at[pl.ds(c, _CHUNK)], sem).start()
        return carry

    lax.fori_loop(0, _NCHUNK, fire, 0)

    def drain(j, carry):
        c = j * _CHUNK
        idxs = idx_v.at[pl.ds(c, _CHUNK)]
        pltpu.make_async_copy(wp_hbm.at[idxs], gp_v.at[pl.ds(c, _CHUNK)], sem).wait()
        pltpu.make_async_copy(wv_hbm.at[idxs], gv_v.at[pl.ds(c, _CHUNK)], sem).wait()
        return carry

    lax.fori_loop(0, _NCHUNK, drain, 0)

    def red_body(g, carry):
        s = g * _L
        acc_p = jnp.zeros((_L,), jnp.float32)
        acc_v = jnp.zeros((_L,), jnp.float32)
        for t in range(_NUM_TILINGS):
            off = t * _SPW + s
            acc_p = acc_p + gp_v[pl.ds(off, _L)]
            acc_v = acc_v + gv_v[pl.ds(off, _L)]
        out_v[pl.ds(s, _L)] = acc_p
        out_v[pl.ds(_SPW + s, _L)] = acc_v
        return carry

    lax.fori_loop(0, _GPW, red_body, 0)

    pltpu.sync_copy(out_v.at[pl.ds(0, _SPW)], out_hbm.at[pl.ds(base, _SPW)])
    pltpu.sync_copy(out_v.at[pl.ds(_SPW, _SPW)],
                    out_hbm.at[pl.ds(_BATCH + base, _SPW)])


def kernel(state, weights_p, weights_v):
    cols = state[:, jnp.array([0, 1, 3, 4])].T.reshape(-1)
    wp = weights_p.reshape(-1)
    wv = weights_v.reshape(-1)
    flat = _tile_coding_sc(cols, wp, wv)
    return flat.reshape(2, _BATCH).T


# speedup vs baseline: 2.6574x; 2.6574x over previous
import functools

import numpy as np
import jax
import jax.numpy as jnp
from jax import lax
from jax.experimental import pallas as pl
from jax.experimental.pallas import tpu as pltpu
from jax.experimental.pallas import tpu_sc as plsc

_NUM_BINS = 512
_NUM_TILINGS = 16
_BATCH = 16384

_NC = 2
_NS = 16
_L = 16
_NW = _NC * _NS
_SPW = _BATCH // _NW
_GPW = _SPW // _L
_IDX_PER_W = _SPW * _NUM_TILINGS
_CHUNK = 128
_NCHUNK = _IDX_PER_W // _CHUNK

_W = np.float32(6.0) / np.float32(_NUM_BINS)
_RW = np.float32(1.0) / _W
_OFFS = [np.float32(np.float32(t) / np.float32(_NUM_TILINGS)) * _W
         for t in range(_NUM_TILINGS)]

_TSTRIDE = _NUM_BINS * _NUM_BINS
_ASTRIDE = _NUM_TILINGS * _TSTRIDE

_mesh = plsc.VectorSubcoreMesh(core_axis_name="c", subcore_axis_name="s")


@functools.partial(
    pl.kernel,
    mesh=_mesh,
    out_type=jax.ShapeDtypeStruct((2 * _BATCH,), jnp.float32),
    scratch_types=[
        pltpu.VMEM((4 * _SPW,), jnp.float32),
        pltpu.VMEM((_IDX_PER_W,), jnp.int32),
        pltpu.VMEM((_IDX_PER_W,), jnp.float32),
        pltpu.VMEM((_IDX_PER_W,), jnp.float32),
        pltpu.VMEM((2 * _SPW,), jnp.float32),
        pltpu.SemaphoreType.DMA,
    ],
)
def _tile_coding_sc(cols_hbm, wp_hbm, wv_hbm, out_hbm,
                    cols_v, idx_v, gp_v, gv_v, out_v, sem):
    wid = lax.axis_index("s") * _NC + lax.axis_index("c")
    base = wid * _SPW

    for c in range(4):
        pltpu.sync_copy(cols_hbm.at[pl.ds(c * _BATCH + base, _SPW)],
                        cols_v.at[pl.ds(c * _SPW, _SPW)])

    def idx_body(g, carry):
        s = g * _L
        x0 = cols_v[pl.ds(s, _L)]
        x1 = cols_v[pl.ds(_SPW + s, _L)]
        s3 = cols_v[pl.ds(2 * _SPW + s, _L)]
        s4 = cols_v[pl.ds(3 * _SPW + s, _L)]
        abase = (s3 + 2.0 * s4).astype(jnp.int32) * _ASTRIDE
        d0 = x0 + 3.0
        d1 = x1 + 3.0
        for t in range(_NUM_TILINGS):
            q0 = (d0 + _OFFS[t]) * _RW
            q1 = (d1 + _OFFS[t]) * _RW
            i0 = jnp.clip(q0.astype(jnp.int32), 0, _NUM_BINS - 1)
            i1 = jnp.clip(q1.astype(jnp.int32), 0, _NUM_BINS - 1)
            flat = (abase + (t * _TSTRIDE)
                    + (i0 >> 3) * 4096 + (i1 >> 7) * 1024
                    + (i0 & 7) * 128 + (i1 & 127))
            idx_v[pl.ds(t * _SPW + s, _L)] = flat
        return carry

    lax.fori_loop(0, _GPW, idx_body, 0)

    def fire(j, carry):
        c = j * _CHUNK
        idxs = idx_v.at[pl.ds(c, _CHUNK)]
        pltpu.make_async_copy(wp_hbm.at[idxs], gp_v.at[pl.ds(c, _CHUNK)], sem).start()
        pltpu.make_async_copy(wv_hbm.at[idxs], gv_v.at[pl.ds(c, _CHUNK)], sem).start()
        return carry

    lax.fori_loop(0, _NCHUNK, fire, 0)

    def drain(j, carry):
        c = j * _CHUNK
        idxs = idx_v.at[pl.ds(c, _CHUNK)]
        pltpu.make_async_copy(wp_hbm.at[idxs], gp_v.at[pl.ds(c, _CHUNK)], sem).wait()
        pltpu.make_async_copy(wv_hbm.at[idxs], gv_v.at[pl.ds(c, _CHUNK)], sem).wait()
        return carry

    lax.fori_loop(0, _NCHUNK, drain, 0)

    def red_body(g, carry):
        s = g * _L
        acc_p = jnp.zeros((_L,), jnp.float32)
        acc_v = jnp.zeros((_L,), jnp.float32)
        for t in range(_NUM_TILINGS):
            off = t * _SPW + s
            acc_p = acc_p + gp_v[pl.ds(off, _L)]
            acc_v = acc_v + gv_v[pl.ds(off, _L)]
        out_v[pl.ds(s, _L)] = acc_p
        out_v[pl.ds(_SPW + s, _L)] = acc_v
        return carry

    lax.fori_loop(0, _GPW, red_body, 0)

    pltpu.sync_copy(out_v.at[pl.ds(0, _SPW)], out_hbm.at[pl.ds(base, _SPW)])
    pltpu.sync_copy(out_v.at[pl.ds(_SPW, _SPW)],
                    out_hbm.at[pl.ds(_BATCH + base, _SPW)])


def kernel(state, weights_p, weights_v):
    cols = state[:, jnp.array([0, 1, 3, 4])].T.reshape(-1)
    def _tiled_view(w):
        return w.reshape(3, _NUM_TILINGS, _NUM_BINS // 8, 8,
                         _NUM_BINS // 128, 128)\
                .transpose(0, 1, 2, 4, 3, 5).reshape(-1)
    wp = _tiled_view(weights_p)
    wv = _tiled_view(weights_v)
    flat = _tile_coding_sc(cols, wp, wv)
    return flat.reshape(2, _BATCH).T


# speedup vs baseline: 2.6837x; 1.0099x over previous
import functools

import numpy as np
import jax
import jax.numpy as jnp
from jax import lax
from jax.experimental import pallas as pl
from jax.experimental.pallas import tpu as pltpu
from jax.experimental.pallas import tpu_sc as plsc

_NUM_BINS = 512
_NUM_TILINGS = 16
_BATCH = 16384

_NC = 2
_NS = 16
_L = 16
_NW = _NC * _NS
_SPW = _BATCH // _NW
_GPW = _SPW // _L
_ROWS = _SPW // 128

_W = np.float32(6.0) / np.float32(_NUM_BINS)
_RW = np.float32(1.0) / _W
_OFFS = [np.float32(np.float32(t) / np.float32(_NUM_TILINGS)) * _W
         for t in range(_NUM_TILINGS)]

_TSTRIDE = _NUM_BINS * _NUM_BINS
_ASTRIDE = _NUM_TILINGS * _TSTRIDE

_mesh = plsc.VectorSubcoreMesh(core_axis_name="c", subcore_axis_name="s")


@functools.partial(
    pl.kernel,
    mesh=_mesh,
    out_type=jax.ShapeDtypeStruct((2, _NW, _SPW), jnp.float32),
    scratch_types=[
        pltpu.VMEM((4 * _SPW,), jnp.float32),
        pltpu.VMEM((_SPW,), jnp.float32),
        pltpu.VMEM((_SPW,), jnp.float32),
        pltpu.VMEM((_SPW,), jnp.int32),
        pltpu.VMEM((_NUM_TILINGS * _SPW,), jnp.int32),
        pltpu.VMEM((_NUM_TILINGS * _SPW,), jnp.float32),
        pltpu.VMEM((_NUM_TILINGS * _SPW,), jnp.float32),
        pltpu.VMEM((_SPW,), jnp.float32),
        pltpu.VMEM((_SPW,), jnp.float32),
        pltpu.SemaphoreType.DMA((_NUM_TILINGS,)),
    ],
)
def _tile_coding_sc(cols_hbm, wp_hbm, wv_hbm, out_hbm,
                    cols_v, d0_v, d1_v, ab_v, idx_v, gp_v, gv_v,
                    accp_v, accv_v, sems):
    wid = lax.axis_index("s") * _NC + lax.axis_index("c")
    base = wid * _SPW

    for c in range(4):
        pltpu.sync_copy(cols_hbm.at[pl.ds(c * _BATCH + base, _SPW)],
                        cols_v.at[pl.ds(c * _SPW, _SPW)])

    def pre_body(g, carry):
        s = g * _L
        x0 = cols_v[pl.ds(s, _L)]
        x1 = cols_v[pl.ds(_SPW + s, _L)]
        s3 = cols_v[pl.ds(2 * _SPW + s, _L)]
        s4 = cols_v[pl.ds(3 * _SPW + s, _L)]
        d0_v[pl.ds(s, _L)] = x0 + 3.0
        d1_v[pl.ds(s, _L)] = x1 + 3.0
        ab_v[pl.ds(s, _L)] = (s3 + 2.0 * s4).astype(jnp.int32) * _ASTRIDE
        return carry

    lax.fori_loop(0, _GPW, pre_body, 0)

    for t in range(_NUM_TILINGS):
        def idx_body(g, carry, _t=t):
            s = g * _L
            d0 = d0_v[pl.ds(s, _L)]
            d1 = d1_v[pl.ds(s, _L)]
            ab = ab_v[pl.ds(s, _L)]
            q0 = (d0 + _OFFS[_t]) * _RW
            q1 = (d1 + _OFFS[_t]) * _RW
            i0 = jnp.clip(q0.astype(jnp.int32), 0, _NUM_BINS - 1)
            i1 = jnp.clip(q1.astype(jnp.int32), 0, _NUM_BINS - 1)
            flat = (ab + (_t * _TSTRIDE)
                    + (i0 >> 3) * 4096 + (i1 >> 7) * 1024
                    + (i0 & 7) * 128 + (i1 & 127))
            idx_v[pl.ds(_t * _SPW + s, _L)] = flat
            return carry

        lax.fori_loop(0, _GPW, idx_body, 0)
        sl = pl.ds(t * _SPW, _SPW)
        pltpu.make_async_copy(wp_hbm.at[idx_v.at[sl]], gp_v.at[sl], sems.at[t]).start()
        pltpu.make_async_copy(wv_hbm.at[idx_v.at[sl]], gv_v.at[sl], sems.at[t]).start()

    for t in range(_NUM_TILINGS):
        sl = pl.ds(t * _SPW, _SPW)
        pltpu.make_async_copy(wp_hbm.at[idx_v.at[sl]], gp_v.at[sl], sems.at[t]).wait()
        pltpu.make_async_copy(wv_hbm.at[idx_v.at[sl]], gv_v.at[sl], sems.at[t]).wait()

        def acc_body(g, carry, _t=t):
            s = g * _L
            p = gp_v[pl.ds(_t * _SPW + s, _L)]
            v = gv_v[pl.ds(_t * _SPW + s, _L)]
            if _t == 0:
                accp_v[pl.ds(s, _L)] = p
                accv_v[pl.ds(s, _L)] = v
            else:
                accp_v[pl.ds(s, _L)] = accp_v[pl.ds(s, _L)] + p
                accv_v[pl.ds(s, _L)] = accv_v[pl.ds(s, _L)] + v
            return carry

        lax.fori_loop(0, _GPW, acc_body, 0)

    pltpu.sync_copy(accp_v, out_hbm.at[0, wid])
    pltpu.sync_copy(accv_v, out_hbm.at[1, wid])


def kernel(state, weights_p, weights_v):
    cols = state[:, jnp.array([0, 1, 3, 4])].T.reshape(-1)

    def _tiled_view(w):
        return w.reshape(3, _NUM_TILINGS, _NUM_BINS // 8, 8,
                         _NUM_BINS // 128, 128)\
                .transpose(0, 1, 2, 4, 3, 5).reshape(-1)

    res = _tile_coding_sc(cols, _tiled_view(weights_p), _tiled_view(weights_v))
    return res.reshape(2, _BATCH).T


# speedup vs baseline: 2.8979x; 1.0798x over previous
import functools

import numpy as np
import jax
import jax.numpy as jnp
from jax import lax
from jax.experimental import pallas as pl
from jax.experimental.pallas import tpu as pltpu
from jax.experimental.pallas import tpu_sc as plsc

_NUM_BINS = 512
_NUM_TILINGS = 16
_BATCH = 16384

_NC = 2
_NS = 16
_L = 16
_NW = _NC * _NS
_SPW = _BATCH // _NW
_GPW = _SPW // _L

_W = np.float32(6.0) / np.float32(_NUM_BINS)
_RW = np.float32(1.0) / _W
_OFFS = [np.float32(np.float32(t) / np.float32(_NUM_TILINGS)) * _W
         for t in range(_NUM_TILINGS)]

_TSTRIDE = _NUM_BINS * _NUM_BINS
_ASTRIDE = _NUM_TILINGS * _TSTRIDE

_mesh = plsc.VectorSubcoreMesh(core_axis_name="c", subcore_axis_name="s")


@functools.partial(
    pl.kernel,
    mesh=_mesh,
    out_type=jax.ShapeDtypeStruct((2, _BATCH), jnp.float32),
    scratch_types=[
        pltpu.VMEM((4 * _SPW,), jnp.float32),
        pltpu.VMEM((_SPW,), jnp.float32),
        pltpu.VMEM((_SPW,), jnp.float32),
        pltpu.VMEM((_SPW,), jnp.int32),
        pltpu.VMEM((_NUM_TILINGS * _SPW,), jnp.int32),
        pltpu.VMEM((_NUM_TILINGS * _SPW,), jnp.float32),
        pltpu.VMEM((_NUM_TILINGS * _SPW,), jnp.float32),
        pltpu.VMEM((_SPW,), jnp.float32),
        pltpu.VMEM((_SPW,), jnp.float32),
        pltpu.SemaphoreType.DMA((_NUM_TILINGS,)),
    ],
)
def _tile_coding_sc(cols_hbm, wp_hbm, wv_hbm, out_hbm,
                    cols_v, d0_v, d1_v, ab_v, idx_v, gp_v, gv_v,
                    accp_v, accv_v, sems):
    wid = lax.axis_index("s") * _NC + lax.axis_index("c")
    base = wid * _SPW

    for c in range(4):
        pltpu.sync_copy(cols_hbm.at[pl.ds(c * _BATCH + base, _SPW)],
                        cols_v.at[pl.ds(c * _SPW, _SPW)])

    zeros = jnp.zeros((_L,), jnp.float32)

    def pre_body(g, carry):
        s = g * _L
        x0 = cols_v[pl.ds(s, _L)]
        x1 = cols_v[pl.ds(_SPW + s, _L)]
        s3 = cols_v[pl.ds(2 * _SPW + s, _L)]
        s4 = cols_v[pl.ds(3 * _SPW + s, _L)]
        d0_v[pl.ds(s, _L)] = x0 + 3.0
        d1_v[pl.ds(s, _L)] = x1 + 3.0
        ab_v[pl.ds(s, _L)] = (s3 + 2.0 * s4).astype(jnp.int32) * _ASTRIDE
        accp_v[pl.ds(s, _L)] = zeros
        accv_v[pl.ds(s, _L)] = zeros
        return carry

    lax.fori_loop(0, _GPW, pre_body, 0)

    def fire_body(t, carry):
        toff = t * _SPW
        o = t.astype(jnp.float32) * jnp.float32(_OFFS[1])
        tbase = t * _TSTRIDE

        def idx_body(g, carry2):
            s = g * _L
            d0 = d0_v[pl.ds(s, _L)]
            d1 = d1_v[pl.ds(s, _L)]
            ab = ab_v[pl.ds(s, _L)]
            q0 = (d0 + o) * _RW
            q1 = (d1 + o) * _RW
            i0 = jnp.clip(q0.astype(jnp.int32), 0, _NUM_BINS - 1)
            i1 = jnp.clip(q1.astype(jnp.int32), 0, _NUM_BINS - 1)
            flat = (ab + tbase
                    + (i0 >> 3) * 4096 + (i1 >> 7) * 1024
                    + (i0 & 7) * 128 + (i1 & 127))
            idx_v[pl.ds(toff + s, _L)] = flat
            return carry2

        lax.fori_loop(0, _GPW, idx_body, 0)
        sl = pl.ds(toff, _SPW)
        pltpu.make_async_copy(wp_hbm.at[idx_v.at[sl]], gp_v.at[sl],
                              sems.at[t]).start()
        pltpu.make_async_copy(wv_hbm.at[idx_v.at[sl]], gv_v.at[sl],
                              sems.at[t]).start()
        return carry

    lax.fori_loop(0, _NUM_TILINGS, fire_body, 0)

    def drain_body(t, carry):
        toff = t * _SPW
        sl = pl.ds(toff, _SPW)
        pltpu.make_async_copy(wp_hbm.at[idx_v.at[sl]], gp_v.at[sl],
                              sems.at[t]).wait()
        pltpu.make_async_copy(wv_hbm.at[idx_v.at[sl]], gv_v.at[sl],
                              sems.at[t]).wait()

        def acc_body(g, carry2):
            s = g * _L
            accp_v[pl.ds(s, _L)] = (accp_v[pl.ds(s, _L)]
                                    + gp_v[pl.ds(toff + s, _L)])
            accv_v[pl.ds(s, _L)] = (accv_v[pl.ds(s, _L)]
                                    + gv_v[pl.ds(toff + s, _L)])
            return carry2

        lax.fori_loop(0, _GPW, acc_body, 0)
        return carry

    lax.fori_loop(0, _NUM_TILINGS, drain_body, 0)

    pltpu.sync_copy(accp_v, out_hbm.at[0, pl.ds(base, _SPW)])
    pltpu.sync_copy(accv_v, out_hbm.at[1, pl.ds(base, _SPW)])


def kernel(state, weights_p, weights_v):
    cols = jnp.concatenate(
        [state[:, 0], state[:, 1], state[:, 3], state[:, 4]])

    def _tiled_view(w):
        return w.reshape(3, _NUM_TILINGS, _NUM_BINS // 8, 8,
                         _NUM_BINS // 128, 128)\
                .transpose(0, 1, 2, 4, 3, 5).reshape(-1)

    res = _tile_coding_sc(cols, _tiled_view(weights_p), _tiled_view(weights_v))
    return res.T


# speedup vs baseline: 2.9044x; 1.0022x over previous
import functools

import numpy as np
import jax
import jax.numpy as jnp
from jax import lax
from jax.experimental import pallas as pl
from jax.experimental.pallas import tpu as pltpu
from jax.experimental.pallas import tpu_sc as plsc

_NUM_BINS = 512
_NUM_TILINGS = 16
_BATCH = 16384

_NC = 2
_NS = 16
_L = 16
_NW = _NC * _NS
_SPW = _BATCH // _NW
_GPW = _SPW // _L

_W = np.float32(6.0) / np.float32(_NUM_BINS)
_RW = np.float32(1.0) / _W
_OFFS = [np.float32(np.float32(t) / np.float32(_NUM_TILINGS)) * _W
         for t in range(_NUM_TILINGS)]

_TSTRIDE = _NUM_BINS * _NUM_BINS
_ASTRIDE = _NUM_TILINGS * _TSTRIDE

_mesh = plsc.VectorSubcoreMesh(core_axis_name="c", subcore_axis_name="s")


@functools.partial(
    pl.kernel,
    mesh=_mesh,
    out_type=jax.ShapeDtypeStruct((2, _BATCH), jnp.float32),
    scratch_types=[
        pltpu.VMEM((4 * _SPW,), jnp.float32),
        pltpu.VMEM((_SPW,), jnp.float32),
        pltpu.VMEM((_SPW,), jnp.float32),
        pltpu.VMEM((_SPW,), jnp.int32),
        pltpu.VMEM((_NUM_TILINGS * _SPW,), jnp.int32),
        pltpu.VMEM((_NUM_TILINGS * _SPW,), jnp.float32),
        pltpu.VMEM((_NUM_TILINGS * _SPW,), jnp.float32),
        pltpu.VMEM((_SPW,), jnp.float32),
        pltpu.VMEM((_SPW,), jnp.float32),
        pltpu.SemaphoreType.DMA((_NUM_TILINGS,)),
    ],
)
def _tile_coding_sc(cols_hbm, wp_hbm, wv_hbm, out_hbm,
                    cols_v, d0_v, d1_v, ab_v, idx_v, gp_v, gv_v,
                    accp_v, accv_v, sems):
    wid = lax.axis_index("s") * _NC + lax.axis_index("c")
    base = wid * _SPW

    for c in range(4):
        pltpu.sync_copy(cols_hbm.at[pl.ds(c * _BATCH + base, _SPW)],
                        cols_v.at[pl.ds(c * _SPW, _SPW)])

    def pre_body(g, carry):
        s = g * _L
        x0 = cols_v[pl.ds(s, _L)]
        x1 = cols_v[pl.ds(_SPW + s, _L)]
        s3 = cols_v[pl.ds(2 * _SPW + s, _L)]
        s4 = cols_v[pl.ds(3 * _SPW + s, _L)]
        d0_v[pl.ds(s, _L)] = x0 + 3.0
        d1_v[pl.ds(s, _L)] = x1 + 3.0
        ab_v[pl.ds(s, _L)] = (s3 + 2.0 * s4).astype(jnp.int32) * _ASTRIDE
        return carry

    lax.fori_loop(0, _GPW, pre_body, 0)

    def fire_body(t, carry):
        toff = t * _SPW
        o = t.astype(jnp.float32) * jnp.float32(_OFFS[1])
        tbase = t * _TSTRIDE

        def idx_body(h, carry2):
            for u in range(2):
                s = h * (2 * _L) + u * _L
                d0 = d0_v[pl.ds(s, _L)]
                d1 = d1_v[pl.ds(s, _L)]
                ab = ab_v[pl.ds(s, _L)]
                q0 = (d0 + o) * _RW
                q1 = (d1 + o) * _RW
                i0 = jnp.clip(q0.astype(jnp.int32), 0, _NUM_BINS - 1)
                i1 = jnp.clip(q1.astype(jnp.int32), 0, _NUM_BINS - 1)
                flat = (ab + tbase
                        + (i0 >> 3) * 4096 + (i1 >> 7) * 1024
                        + (i0 & 7) * 128 + (i1 & 127))
                idx_v[pl.ds(toff + s, _L)] = flat
            return carry2

        lax.fori_loop(0, _GPW // 2, idx_body, 0)
        sl = pl.ds(toff, _SPW)
        pltpu.make_async_copy(wp_hbm.at[idx_v.at[sl]], gp_v.at[sl],
                              sems.at[t]).start()
        pltpu.make_async_copy(wv_hbm.at[idx_v.at[sl]], gv_v.at[sl],
                              sems.at[t]).start()
        return carry

    lax.fori_loop(0, _NUM_TILINGS, fire_body, 0)

    sl0 = pl.ds(0, _SPW)
    pltpu.make_async_copy(wp_hbm.at[idx_v.at[sl0]], gp_v.at[sl0],
                          sems.at[0]).wait()
    pltpu.make_async_copy(wv_hbm.at[idx_v.at[sl0]], gv_v.at[sl0],
                          sems.at[0]).wait()

    def init_body(h, carry):
        for u in range(2):
            s = h * (2 * _L) + u * _L
            accp_v[pl.ds(s, _L)] = gp_v[pl.ds(s, _L)]
            accv_v[pl.ds(s, _L)] = gv_v[pl.ds(s, _L)]
        return carry

    lax.fori_loop(0, _GPW // 2, init_body, 0)

    def drain_body(t, carry):
        toff = t * _SPW
        sl = pl.ds(toff, _SPW)
        pltpu.make_async_copy(wp_hbm.at[idx_v.at[sl]], gp_v.at[sl],
                              sems.at[t]).wait()
        pltpu.make_async_copy(wv_hbm.at[idx_v.at[sl]], gv_v.at[sl],
                              sems.at[t]).wait()

        def acc_body(h, carry2):
            for u in range(2):
                s = h * (2 * _L) + u * _L
                accp_v[pl.ds(s, _L)] = (accp_v[pl.ds(s, _L)]
                                        + gp_v[pl.ds(toff + s, _L)])
                accv_v[pl.ds(s, _L)] = (accv_v[pl.ds(s, _L)]
                                        + gv_v[pl.ds(toff + s, _L)])
            return carry2

        lax.fori_loop(0, _GPW // 2, acc_body, 0)
        return carry

    lax.fori_loop(1, _NUM_TILINGS, drain_body, 0)

    pltpu.sync_copy(accp_v, out_hbm.at[0, pl.ds(base, _SPW)])
    pltpu.sync_copy(accv_v, out_hbm.at[1, pl.ds(base, _SPW)])


def kernel(state, weights_p, weights_v):
    cols = jnp.concatenate(
        [state[:, 0], state[:, 1], state[:, 3], state[:, 4]])

    def _tiled_view(w):
        return w.reshape(3, _NUM_TILINGS, _NUM_BINS // 8, 8,
                         _NUM_BINS // 128, 128)\
                .transpose(0, 1, 2, 4, 3, 5).reshape(-1)

    res = _tile_coding_sc(cols, _tiled_view(weights_p), _tiled_view(weights_v))
    return res.T
